# baseline (speedup 1.0000x reference)
import functools

import jax
import jax.numpy as jnp
from jax import lax
from jax.experimental import pallas as pl
from jax.experimental.pallas import tpu as pltpu
from jax.experimental.pallas import tpu_sc as plsc

_TOPK = 6
_N_EXPERTS = 64
_DIM = 2048
_N_TOKENS = 32768

_NC = 2
_NS = 16
_NW = _NC * _NS
_NCHUNK = 1
_CHUNK = _N_TOKENS // _NCHUNK
_TPW = _CHUNK // _NW
_LANES = 16
_GROUPS = _TPW // _LANES


def _tc_probs_body(x_ref, w_ref, out_ref):
    scores = lax.dot_general(
        w_ref[...], x_ref[...],
        dimension_numbers=(((1,), (1,)), ((), ())),
        preferred_element_type=jnp.float32,
    )
    m = jnp.max(scores, axis=0, keepdims=True)
    e = jnp.exp(scores - m)
    s = jnp.sum(e, axis=0, keepdims=True)
    probs = e / s
    pi = lax.bitcast_convert_type(probs, jnp.int32) + jnp.int32(32)
    inv = jnp.int32(63) - lax.broadcasted_iota(
        jnp.int32, (_N_EXPERTS, _TPW), 0)
    out_ref[0] = lax.bitcast_convert_type(
        (pi & jnp.int32(~63)) | inv, jnp.float32)


def _tc_probs(x, W, *, interpret=False):
    return pl.pallas_call(
        _tc_probs_body,
        grid=(x.shape[0] // _TPW,),
        in_specs=[
            pl.BlockSpec((_TPW, _DIM), lambda i: (i, 0)),
            pl.BlockSpec((_N_EXPERTS, _DIM), lambda i: (0, 0)),
        ],
        out_specs=pl.BlockSpec((1, _N_EXPERTS, _TPW), lambda i: (i, 0, 0)),
        out_shape=jax.ShapeDtypeStruct(
            (x.shape[0] // _TPW, _N_EXPERTS, _TPW), jnp.float32),
        interpret=interpret,
    )(x, W)


def _sc_top6_body(probs_hbm, w_hbm, i_hbm, buf, wbuf, ibuf):
    wid = lax.axis_index("s") * _NC + lax.axis_index("c")
    pltpu.sync_copy(probs_hbm.at[wid], buf)

    def group_step(g, _):
        base = g * _LANES

        def expert_step(e, ws):
            v = buf[e, pl.ds(base, _LANES)]
            ws = list(ws)
            for j in range(_TOPK):
                ws[j], v = jnp.maximum(v, ws[j]), jnp.minimum(v, ws[j])
            return tuple(ws)

        init = tuple(jnp.full((_LANES,), -1.0, jnp.float32)
                     for _ in range(_TOPK))
        ws = lax.fori_loop(0, _N_EXPERTS, expert_step, init, unroll=16)
        for j in range(_TOPK):
            pj = lax.bitcast_convert_type(ws[j], jnp.int32)
            wbuf[j, pl.ds(base, _LANES)] = lax.bitcast_convert_type(
                pj & jnp.int32(~63), jnp.float32)
            ibuf[j, pl.ds(base, _LANES)] = jnp.int32(63) - (pj & jnp.int32(63))
        return 0

    lax.fori_loop(0, _GROUPS, group_step, 0)
    pltpu.sync_copy(wbuf, w_hbm.at[wid])
    pltpu.sync_copy(ibuf, i_hbm.at[wid])


def _sc_top6(probs, *, interpret=False):
    mesh = plsc.VectorSubcoreMesh(core_axis_name="c", subcore_axis_name="s")
    f = functools.partial(
        pl.kernel,
        out_type=(
            jax.ShapeDtypeStruct((_NW, _TOPK, _TPW), jnp.float32),
            jax.ShapeDtypeStruct((_NW, _TOPK, _TPW), jnp.int32),
        ),
        mesh=mesh,
        scratch_types=[
            pltpu.VMEM((_N_EXPERTS, _TPW), jnp.float32),
            pltpu.VMEM((_TOPK, _TPW), jnp.float32),
            pltpu.VMEM((_TOPK, _TPW), jnp.int32),
        ],
        interpret=interpret,
    )(_sc_top6_body)
    return f(probs)


def kernel(x, W):
    assert x.shape == (_N_TOKENS, _DIM)
    assert W.shape == (_N_EXPERTS, _DIM)
    probs = _tc_probs(x, W)
    w_t, i_t = _sc_top6(probs)
    weights = w_t.transpose(0, 2, 1).reshape(_N_TOKENS, _TOPK)
    indices = i_t.transpose(0, 2, 1).reshape(_N_TOKENS, _TOPK)
    return weights, indices


# speedup vs baseline: 1.0094x; 1.0094x over previous
import functools

import jax
import jax.numpy as jnp
from jax import lax
from jax.experimental import pallas as pl
from jax.experimental.pallas import tpu as pltpu
from jax.experimental.pallas import tpu_sc as plsc

_TOPK = 6
_N_EXPERTS = 64
_DIM = 2048
_N_TOKENS = 32768

_NC = 2
_NS = 16
_NW = _NC * _NS
_NCHUNK = 1
_CHUNK = _N_TOKENS // _NCHUNK
_TPW = _CHUNK // _NW
_LANES = 16
_GROUPS = _TPW // _LANES


_TB = 2048
_TBR = _TB // _TPW


def _tc_probs_body(x_ref, w_ref, out_ref):
    scores = lax.dot_general(
        w_ref[...], x_ref[...],
        dimension_numbers=(((1,), (1,)), ((), ())),
        preferred_element_type=jnp.float32,
    )
    m = jnp.max(scores, axis=0, keepdims=True)
    e = jnp.exp(scores - m)
    s = jnp.sum(e, axis=0, keepdims=True)
    probs = e / s
    pi = lax.bitcast_convert_type(probs, jnp.int32) + jnp.int32(32)
    inv = jnp.int32(63) - lax.broadcasted_iota(
        jnp.int32, (_N_EXPERTS, _TB), 0)
    packed = lax.bitcast_convert_type(
        (pi & jnp.int32(~63)) | inv, jnp.float32)
    for r in range(_TBR):
        out_ref[r] = packed[:, r * _TPW:(r + 1) * _TPW]


def _tc_probs(x, W, *, interpret=False):
    return pl.pallas_call(
        _tc_probs_body,
        grid=(x.shape[0] // _TB,),
        in_specs=[
            pl.BlockSpec((_TB, _DIM), lambda i: (i, 0)),
            pl.BlockSpec((_N_EXPERTS, _DIM), lambda i: (0, 0)),
        ],
        out_specs=pl.BlockSpec((_TBR, _N_EXPERTS, _TPW), lambda i: (i, 0, 0)),
        out_shape=jax.ShapeDtypeStruct(
            (x.shape[0] // _TPW, _N_EXPERTS, _TPW), jnp.float32),
        interpret=interpret,
    )(x, W)


def _sc_top6_body(probs_hbm, w_hbm, i_hbm, buf, wbuf, ibuf):
    wid = lax.axis_index("s") * _NC + lax.axis_index("c")
    pltpu.sync_copy(probs_hbm.at[wid], buf)

    def group_step(g, _):
        base = g * _LANES

        def expert_step(e, ws):
            v = buf[e, pl.ds(base, _LANES)]
            ws = list(ws)
            for j in range(_TOPK):
                ws[j], v = jnp.maximum(v, ws[j]), jnp.minimum(v, ws[j])
            return tuple(ws)

        init = tuple(jnp.full((_LANES,), -1.0, jnp.float32)
                     for _ in range(_TOPK))
        ws = lax.fori_loop(0, _N_EXPERTS, expert_step, init, unroll=16)
        for j in range(_TOPK):
            pj = lax.bitcast_convert_type(ws[j], jnp.int32)
            wbuf[j, pl.ds(base, _LANES)] = lax.bitcast_convert_type(
                pj & jnp.int32(~63), jnp.float32)
            ibuf[j, pl.ds(base, _LANES)] = jnp.int32(63) - (pj & jnp.int32(63))
        return 0

    lax.fori_loop(0, _GROUPS, group_step, 0)
    pltpu.sync_copy(wbuf, w_hbm.at[wid])
    pltpu.sync_copy(ibuf, i_hbm.at[wid])


def _sc_top6(probs, *, interpret=False):
    mesh = plsc.VectorSubcoreMesh(core_axis_name="c", subcore_axis_name="s")
    f = functools.partial(
        pl.kernel,
        out_type=(
            jax.ShapeDtypeStruct((_NW, _TOPK, _TPW), jnp.float32),
            jax.ShapeDtypeStruct((_NW, _TOPK, _TPW), jnp.int32),
        ),
        mesh=mesh,
        scratch_types=[
            pltpu.VMEM((_N_EXPERTS, _TPW), jnp.float32),
            pltpu.VMEM((_TOPK, _TPW), jnp.float32),
            pltpu.VMEM((_TOPK, _TPW), jnp.int32),
        ],
        interpret=interpret,
    )(_sc_top6_body)
    return f(probs)


def kernel(x, W):
    assert x.shape == (_N_TOKENS, _DIM)
    assert W.shape == (_N_EXPERTS, _DIM)
    probs = _tc_probs(x, W)
    w_t, i_t = _sc_top6(probs)
    weights = w_t.transpose(0, 2, 1).reshape(_N_TOKENS, _TOPK)
    indices = i_t.transpose(0, 2, 1).reshape(_N_TOKENS, _TOPK)
    return weights, indices
